# baseline (speedup 1.0000x reference)
import functools

import jax
import jax.numpy as jnp
from jax import lax
from jax.experimental import pallas as pl
from jax.experimental.pallas import tpu as pltpu
from jax.experimental.pallas import tpu_sc as plsc

_B = 4
_S = 8192
_D = 768
_NW = 32
_STRIP = _S // _NW
_CH = 16
_NCHUNK = _STRIP // _CH
_NI = _NCHUNK // 2
_LANES = 16


def _sc_add(x_hbm, emb_hbm, out_hbm, xv, ev,
            sx0, sx1, sx2, sx3, se0, se1, so0, so1, so2, so3):
    sx = [sx0, sx1, sx2, sx3]
    se = [se0, se1]
    so = [so0, so1, so2, so3]
    wid = lax.axis_index("s") * 2 + lax.axis_index("c")
    seq0 = wid * _STRIP

    def e_start(c, par):
        pltpu.async_copy(
            emb_hbm.at[pl.ds(seq0 + c * _CH, _CH)], ev.at[par], se[par]
        )

    def e_wait(par):
        pltpu.make_async_copy(
            emb_hbm.at[pl.ds(0, _CH)], ev.at[par], se[par]
        ).wait()

    def x_start(c, b, buf):
        pltpu.async_copy(
            x_hbm.at[pl.ds(b * _S + seq0 + c * _CH, _CH)],
            xv.at[buf],
            sx[buf],
        )

    def x_wait(buf):
        pltpu.make_async_copy(
            x_hbm.at[pl.ds(0, _CH)], xv.at[buf], sx[buf]
        ).wait()

    def out_start(c, b, buf):
        pltpu.async_copy(
            xv.at[buf],
            out_hbm.at[pl.ds(b * _S + seq0 + c * _CH, _CH)],
            so[buf],
        )

    def out_wait(buf):
        pltpu.make_async_copy(
            xv.at[buf], out_hbm.at[pl.ds(0, _CH)], so[buf]
        ).wait()

    def compute(buf, par):
        @plsc.parallel_loop(0, _CH, 1, unroll=2)
        def _(r):
            for g in range(_D // _LANES):
                sl = pl.ds(g * _LANES, _LANES)
                plsc.addupdate(xv.at[buf, r, sl], ev[par, r, sl])

    e_start(0, 0)
    x_start(0, 0, 0)

    def iter_body(i, carry):
        for par in range(2):
            c = 2 * i + par
            e_wait(par)
            if par == 0:
                e_start(c + 1, 1)
            else:
                @pl.when(i < _NI - 1)
                def _():
                    e_start(c + 1, 0)

            for b in range(4):
                nbuf = (b + 1) % 4
                if par == 0 and b < 3:
                    @pl.when(i > 0)
                    def _():
                        out_wait(nbuf)
                else:
                    out_wait(nbuf)
                if b < 3:
                    x_start(c, b + 1, nbuf)
                elif par == 0:
                    x_start(c + 1, 0, nbuf)
                else:
                    @pl.when(i < _NI - 1)
                    def _():
                        x_start(c + 1, 0, nbuf)
                x_wait(b % 4)
                compute(b % 4, par)
                out_start(c, b, b % 4)
        return carry

    lax.fori_loop(0, _NI, iter_body, 0)
    for buf in (1, 2, 3):
        out_wait(buf)


@jax.jit
def _sc_kernel(x2d, emb_table):
    mesh = plsc.VectorSubcoreMesh(core_axis_name="c", subcore_axis_name="s")
    return pl.kernel(
        _sc_add,
        out_type=jax.ShapeDtypeStruct((_B * _S, _D), jnp.float32),
        mesh=mesh,
        scratch_types=[
            pltpu.VMEM((4, _CH, _D), jnp.float32),
            pltpu.VMEM((2, _CH, _D), jnp.float32),
        ] + [pltpu.SemaphoreType.DMA] * 10,
        compiler_params=pltpu.CompilerParams(use_tc_tiling_on_sc=True),
    )(x2d, emb_table)


def kernel(x, emb_table):
    B, S, D = x.shape
    out = _sc_kernel(x.reshape(B * S, D), emb_table)
    return out.reshape(B, S, D)


# speedup vs baseline: 1.3628x; 1.3628x over previous
import functools

import jax
import jax.numpy as jnp
from jax import lax
from jax.experimental import pallas as pl
from jax.experimental.pallas import tpu as pltpu
from jax.experimental.pallas import tpu_sc as plsc

_B = 4
_S = 8192
_D = 768
_NW = 32
_STRIP = _S // _NW
_CH = 16
_NCHUNK = _STRIP // _CH
_NI = _NCHUNK // 2
_LANES = 16


def _sc_add(x_hbm, emb_hbm, out_hbm, xv, ev,
            sx0, sx1, sx2, sx3, se0, se1, so0, so1, so2, so3):
    sx = [sx0, sx1, sx2, sx3]
    se = [se0, se1]
    so = [so0, so1, so2, so3]
    wid = lax.axis_index("s") * 2 + lax.axis_index("c")
    seq0 = wid * _STRIP

    def e_start(c, par):
        pltpu.async_copy(
            emb_hbm.at[pl.ds(seq0 + c * _CH, _CH)], ev.at[par], se[par]
        )

    def e_wait(par):
        pltpu.make_async_copy(
            emb_hbm.at[pl.ds(0, _CH)], ev.at[par], se[par]
        ).wait()

    def x_start(c, b, buf):
        pltpu.async_copy(
            x_hbm.at[pl.ds(b * _S + seq0 + c * _CH, _CH)],
            xv.at[buf],
            sx[buf],
        )

    def x_wait(buf):
        pltpu.make_async_copy(
            x_hbm.at[pl.ds(0, _CH)], xv.at[buf], sx[buf]
        ).wait()

    def out_start(c, b, buf):
        pltpu.async_copy(
            xv.at[buf],
            out_hbm.at[pl.ds(b * _S + seq0 + c * _CH, _CH)],
            so[buf],
        )

    def out_wait(buf):
        pltpu.make_async_copy(
            xv.at[buf], out_hbm.at[pl.ds(0, _CH)], so[buf]
        ).wait()

    def compute(buf, par):
        def row_body(r, carry):
            @plsc.parallel_loop(0, _D, _LANES, unroll=16)
            def _(col):
                sl = pl.ds(col, _LANES)
                plsc.addupdate(xv.at[buf, r, sl], ev[par, r, sl])
            return carry

        lax.fori_loop(0, _CH, row_body, 0)

    e_start(0, 0)
    x_start(0, 0, 0)

    def iter_body(i, carry):
        for par in range(2):
            c = 2 * i + par
            e_wait(par)
            if par == 0:
                e_start(c + 1, 1)
            else:
                @pl.when(i < _NI - 1)
                def _():
                    e_start(c + 1, 0)

            for b in range(4):
                nbuf = (b + 1) % 4
                if par == 0 and b < 3:
                    @pl.when(i > 0)
                    def _():
                        out_wait(nbuf)
                else:
                    out_wait(nbuf)
                if b < 3:
                    x_start(c, b + 1, nbuf)
                elif par == 0:
                    x_start(c + 1, 0, nbuf)
                else:
                    @pl.when(i < _NI - 1)
                    def _():
                        x_start(c + 1, 0, nbuf)
                x_wait(b % 4)
                compute(b % 4, par)
                out_start(c, b, b % 4)
        return carry

    lax.fori_loop(0, _NI, iter_body, 0)
    for buf in (1, 2, 3):
        out_wait(buf)


@jax.jit
def _sc_kernel(x2d, emb_table):
    mesh = plsc.VectorSubcoreMesh(core_axis_name="c", subcore_axis_name="s")
    return pl.kernel(
        _sc_add,
        out_type=jax.ShapeDtypeStruct((_B * _S, _D), jnp.float32),
        mesh=mesh,
        scratch_types=[
            pltpu.VMEM((4, _CH, _D), jnp.float32),
            pltpu.VMEM((2, _CH, _D), jnp.float32),
        ] + [pltpu.SemaphoreType.DMA] * 10,
        compiler_params=pltpu.CompilerParams(use_tc_tiling_on_sc=True),
    )(x2d, emb_table)


def kernel(x, emb_table):
    B, S, D = x.shape
    out = _sc_kernel(x.reshape(B * S, D), emb_table)
    return out.reshape(B, S, D)


# speedup vs baseline: 1.3724x; 1.0071x over previous
import functools

import jax
import jax.numpy as jnp
from jax import lax
from jax.experimental import pallas as pl
from jax.experimental.pallas import tpu as pltpu
from jax.experimental.pallas import tpu_sc as plsc

_B = 4
_S = 8192
_D = 768
_NW = 32
_STRIP = _S // _NW
_CH = 16
_NCHUNK = _STRIP // _CH
_NI = _NCHUNK // 2
_LANES = 16


def _sc_add(x_hbm, emb_hbm, out_hbm, xv, ev,
            sx0, sx1, sx2, sx3, se0, se1, so0, so1, so2, so3):
    sx = [sx0, sx1, sx2, sx3]
    se = [se0, se1]
    so = [so0, so1, so2, so3]
    wid = lax.axis_index("s") * 2 + lax.axis_index("c")
    seq0 = wid * _STRIP

    def e_start(c, par):
        pltpu.async_copy(
            emb_hbm.at[pl.ds(seq0 + c * _CH, _CH)], ev.at[par], se[par]
        )

    def e_wait(par):
        pltpu.make_async_copy(
            emb_hbm.at[pl.ds(0, _CH)], ev.at[par], se[par]
        ).wait()

    def x_start(c, b, buf):
        pltpu.async_copy(
            x_hbm.at[pl.ds(b * _S + seq0 + c * _CH, _CH)],
            xv.at[buf],
            sx[buf],
        )

    def x_wait(buf):
        pltpu.make_async_copy(
            x_hbm.at[pl.ds(0, _CH)], xv.at[buf], sx[buf]
        ).wait()

    def out_start(c, b, buf):
        pltpu.async_copy(
            xv.at[buf],
            out_hbm.at[pl.ds(b * _S + seq0 + c * _CH, _CH)],
            so[buf],
        )

    def out_wait(buf):
        pltpu.make_async_copy(
            xv.at[buf], out_hbm.at[pl.ds(0, _CH)], so[buf]
        ).wait()

    def compute(buf, par):
        def row_body(r, carry):
            @plsc.parallel_loop(0, _D, _LANES, unroll=8)
            def _(col):
                sl = pl.ds(col, _LANES)
                plsc.addupdate(xv.at[buf, r, sl], ev[par, r, sl])
            return carry

        lax.fori_loop(0, _CH, row_body, 0)

    e_start(0, 0)
    x_start(0, 0, 0)

    def iter_body(i, carry):
        for par in range(2):
            c = 2 * i + par
            e_wait(par)
            if par == 0:
                e_start(c + 1, 1)
            else:
                @pl.when(i < _NI - 1)
                def _():
                    e_start(c + 1, 0)

            for b in range(4):
                nbuf = (b + 1) % 4
                if par == 0 and b < 3:
                    @pl.when(i > 0)
                    def _():
                        out_wait(nbuf)
                else:
                    out_wait(nbuf)
                if b < 3:
                    x_start(c, b + 1, nbuf)
                elif par == 0:
                    x_start(c + 1, 0, nbuf)
                else:
                    @pl.when(i < _NI - 1)
                    def _():
                        x_start(c + 1, 0, nbuf)
                x_wait(b % 4)
                compute(b % 4, par)
                out_start(c, b, b % 4)
        return carry

    lax.fori_loop(0, _NI, iter_body, 0)
    for buf in (1, 2, 3):
        out_wait(buf)


@jax.jit
def _sc_kernel(x2d, emb_table):
    mesh = plsc.VectorSubcoreMesh(core_axis_name="c", subcore_axis_name="s")
    return pl.kernel(
        _sc_add,
        out_type=jax.ShapeDtypeStruct((_B * _S, _D), jnp.float32),
        mesh=mesh,
        scratch_types=[
            pltpu.VMEM((4, _CH, _D), jnp.float32),
            pltpu.VMEM((2, _CH, _D), jnp.float32),
        ] + [pltpu.SemaphoreType.DMA] * 10,
        compiler_params=pltpu.CompilerParams(use_tc_tiling_on_sc=True),
    )(x2d, emb_table)


def kernel(x, emb_table):
    B, S, D = x.shape
    out = _sc_kernel(x.reshape(B * S, D), emb_table)
    return out.reshape(B, S, D)
